# Initial kernel scaffold; baseline (speedup 1.0000x reference)
import jax
import jax.numpy as jnp
from jax.experimental import pallas as pl


def kernel(features, neighbor_idx, segment_ids, num_samples):
    raise NotImplementedError("write your pallas kernel here")


# speedup vs baseline: 4.7717x; 4.7717x over previous
import functools

import jax
import jax.numpy as jnp
from jax import lax
from jax.experimental import pallas as pl
from jax.experimental.pallas import tpu as pltpu, tpu_sc as plsc

N_NODES = 10000
N_EDGES = 320000
D_FEAT = 128

_NC = 2
_NS = 16
_NW = _NC * _NS
_LANES = 16

_CHUNK = N_EDGES // _NW
_K = 80
_NBATCH = _CHUNK // _K
_N_PAD = 10240
_ROWS_PER_TILE = _N_PAD // _NS

_mesh = plsc.VectorSubcoreMesh(core_axis_name="c", subcore_axis_name="s")


def _fill_2d(ref, nrows, ncols, val):
    v = jnp.full((_LANES,), val, jnp.float32)

    def row(i, _):
        for j in range(ncols // _LANES):
            ref[i, pl.ds(j * _LANES, _LANES)] = v
        return 0

    lax.fori_loop(0, nrows, row, 0)


@functools.partial(
    pl.kernel,
    out_type=(
        jax.ShapeDtypeStruct((_NC, _N_PAD, D_FEAT), jnp.float32),
        jax.ShapeDtypeStruct((_NC, _N_PAD, D_FEAT), jnp.float32),
    ),
    mesh=_mesh,
    scratch_types=(
        pltpu.VMEM((_K,), jnp.int32),
        pltpu.VMEM((_K,), jnp.int32),
        pltpu.VMEM((_K, D_FEAT), jnp.float32),
        pltpu.VMEM((_K, D_FEAT), jnp.float32),
        pltpu.VMEM_SHARED((_N_PAD, D_FEAT), jnp.float32),
        pltpu.SemaphoreType.DMA,
    ),
)
def _phase1(feat_hbm, nidx_hbm, seg_hbm, sums_out, cnts_out,
            idx_v, seg_v, rows_v, ones_v, acc, sem):
    cid = lax.axis_index("c")
    sid = lax.axis_index("s")
    wid = cid * _NS + sid
    base = wid * _CHUNK
    r0 = sid * _ROWS_PER_TILE
    nzb = _ROWS_PER_TILE // _K

    _fill_2d(rows_v, _K, D_FEAT, 0.0)
    for i in range(nzb):
        pltpu.sync_copy(rows_v, acc.at[pl.ds(r0 + i * _K, _K)])
    plsc.subcore_barrier()

    def body_a(i, _):
        b = base + i * _K
        pltpu.sync_copy(nidx_hbm.at[pl.ds(b, _K)], idx_v)
        pltpu.sync_copy(seg_hbm.at[pl.ds(b, _K)], seg_v)
        pltpu.async_copy(feat_hbm.at[idx_v], rows_v, sem).wait()
        pltpu.sync_copy(rows_v, acc.at[seg_v], add=True)
        return 0

    lax.fori_loop(0, _NBATCH, body_a, 0)
    plsc.subcore_barrier()
    pltpu.sync_copy(acc.at[pl.ds(r0, _ROWS_PER_TILE)],
                    sums_out.at[cid, pl.ds(r0, _ROWS_PER_TILE)])
    plsc.subcore_barrier()

    _fill_2d(rows_v, _K, D_FEAT, 0.0)
    _fill_2d(ones_v, _K, D_FEAT, 1.0)
    for i in range(nzb):
        pltpu.sync_copy(rows_v, acc.at[pl.ds(r0 + i * _K, _K)])
    plsc.subcore_barrier()

    def body_b(i, _):
        b = base + i * _K
        pltpu.sync_copy(seg_hbm.at[pl.ds(b, _K)], seg_v)
        pltpu.sync_copy(ones_v, acc.at[seg_v], add=True)
        return 0

    lax.fori_loop(0, _NBATCH, body_b, 0)
    plsc.subcore_barrier()
    pltpu.sync_copy(acc.at[pl.ds(r0, _ROWS_PER_TILE)],
                    cnts_out.at[cid, pl.ds(r0, _ROWS_PER_TILE)])


def _combine_body(sums_ref, cnts_ref, out_ref):
    s = sums_ref[0] + sums_ref[1]
    c = cnts_ref[0] + cnts_ref[1]
    out_ref[...] = jnp.where(c > 0.0, s / jnp.maximum(c, 1.0), 0.0)


_combine = pl.pallas_call(
    _combine_body,
    out_shape=jax.ShapeDtypeStruct((N_NODES, D_FEAT), jnp.float32),
)


def kernel(features, neighbor_idx, segment_ids, num_samples):
    del num_samples
    sums, cnts = _phase1(features, neighbor_idx, segment_ids)
    sums = sums[:, :N_NODES, :]
    cnts = cnts[:, :N_NODES, 0:1]
    return _combine(sums, cnts)
